# baseline (speedup 1.0000x reference)
import functools

import jax
import jax.numpy as jnp
from jax import lax
from jax.experimental import pallas as pl
from jax.experimental.pallas import tpu as pltpu
from jax.experimental.pallas import tpu_sc as plsc

NC = 2
NS = 16
NW = NC * NS
CHUNK = 128

_MESH = plsc.VectorSubcoreMesh(core_axis_name="c", subcore_axis_name="s")


def _pad_edges(edge_index, n_nodes, p_rows):
    e = edge_index.shape[1]
    per = NW * CHUNK * 3
    e_pad = ((e + per - 1) // per) * per
    extra = e_pad - e
    if extra:
        gar = n_nodes + jnp.arange(extra, dtype=jnp.int32) % (p_rows - n_nodes)
        pad = jnp.stack([jnp.zeros((extra,), jnp.int32), gar])
        edge_index = jnp.concatenate([edge_index, pad], axis=1)
    return edge_index.reshape(2, -1, CHUNK)


def _deg_body(ei_hbm, zeros_hbm, out_hbm, acc_sh, dst_sta, ones_v, sem, *,
              t_chunks, p_rows):
    c = lax.axis_index("c")
    s = lax.axis_index("s")
    stripe = p_rows // NS
    row0 = (c * NS + s) * t_chunks
    pltpu.sync_copy(ei_hbm.at[1, pl.ds(row0, t_chunks)], dst_sta)
    for k in range(stripe // CHUNK):
        off = s * stripe + k * CHUNK
        pltpu.sync_copy(zeros_hbm.at[pl.ds(off, CHUNK)], ones_v)
        pltpu.sync_copy(ones_v, acc_sh.at[pl.ds(off, CHUNK)])
    for i in range(CHUNK // 16):
        ones_v[pl.ds(i * 16, 16)] = jnp.ones((16,), jnp.float32)
    plsc.subcore_barrier()

    def body(j, carry):
        pltpu.async_copy(ones_v, acc_sh.at[dst_sta.at[j]], sem, add=True)
        return carry

    lax.fori_loop(0, t_chunks, body, 0)

    def drain(j, carry):
        pltpu.make_async_copy(ones_v, acc_sh.at[dst_sta.at[0]], sem).wait()
        return carry

    lax.fori_loop(0, t_chunks, drain, 0)
    plsc.subcore_barrier()
    zbuf = ones_v
    for k in range(stripe // CHUNK):
        off = s * stripe + k * CHUNK
        pltpu.sync_copy(acc_sh.at[pl.ds(off, CHUNK)], zbuf)
        pltpu.sync_copy(zbuf, out_hbm.at[pl.ds(c * p_rows + off, CHUNK)])


def _deg_sc(ei_pad, zeros1, p_rows):
    t_chunks = ei_pad.shape[1] // NW
    body = functools.partial(_deg_body, t_chunks=t_chunks, p_rows=p_rows)
    return pl.kernel(
        body,
        out_type=jax.ShapeDtypeStruct((NC * p_rows,), jnp.float32),
        mesh=_MESH,
        compiler_params=pltpu.CompilerParams(use_tc_tiling_on_sc=False),
        scratch_types=[
            pltpu.VMEM_SHARED((p_rows,), jnp.float32),
            pltpu.VMEM((t_chunks, CHUNK), jnp.int32),
            pltpu.VMEM((CHUNK,), jnp.float32),
            pltpu.SemaphoreType.DMA,
        ],
    )(ei_pad, zeros1)


def _agg_body(y_hbm, ei_hbm, zeros_hbm, out_hbm, acc_sh, y_sh, src_sta,
              dst_sta, b0, b1, b2, g0, g1, g2, s0, s1, s2, *,
              t_chunks, p_rows, n_rows, width):
    c = lax.axis_index("c")
    s = lax.axis_index("s")
    stripe = p_rows // NS
    row0 = (c * NS + s) * t_chunks
    pltpu.sync_copy(ei_hbm.at[0, pl.ds(row0, t_chunks)], src_sta)
    pltpu.sync_copy(ei_hbm.at[1, pl.ds(row0, t_chunks)], dst_sta)
    for k in range(stripe // CHUNK):
        off = s * stripe + k * CHUNK
        pltpu.sync_copy(zeros_hbm.at[pl.ds(off, CHUNK)], b0)
        pltpu.sync_copy(b0, acc_sh.at[pl.ds(off, CHUNK)])
    y_stripe = n_rows // NS
    for k in range((y_stripe + CHUNK - 1) // CHUNK):
        off = s * y_stripe + k * CHUNK
        cnt = min(CHUNK, y_stripe - k * CHUNK)
        pltpu.sync_copy(y_hbm.at[pl.ds(off, cnt)], b1.at[pl.ds(0, cnt)])
        pltpu.sync_copy(b1.at[pl.ds(0, cnt)], y_sh.at[pl.ds(off, cnt)])
    plsc.subcore_barrier()

    bufs = (b0, b1, b2)
    gsem = (g0, g1, g2)
    ssem = (s0, s1, s2)

    def gather(q, b):
        src = y_hbm if b == 0 else y_sh
        pltpu.async_copy(src.at[src_sta.at[q]], bufs[b], gsem[b])

    def gather_wait(b):
        pltpu.make_async_copy(y_sh.at[src_sta.at[0]], bufs[b],
                              gsem[b]).wait()

    def scatter(q, b):
        pltpu.async_copy(bufs[b], acc_sh.at[dst_sta.at[q]], ssem[b],
                         add=True)

    def scatter_wait(b):
        pltpu.make_async_copy(bufs[b], acc_sh.at[dst_sta.at[0]],
                              ssem[b]).wait()

    gather(0, 0)
    gather(1, 1)

    def body(i, carry):
        for b3 in range(3):
            q = i * 3 + b3
            nb = (b3 + 2) % 3

            @pl.when(q >= 1)
            def _():
                scatter_wait(nb)

            @pl.when(q + 2 < t_chunks)
            def _():
                gather(q + 2, nb)

            gather_wait(b3)
            scatter(q, b3)
        return carry

    lax.fori_loop(0, t_chunks // 3, body, 0)
    scatter_wait((t_chunks - 1) % 3)
    plsc.subcore_barrier()
    for k in range(stripe // CHUNK):
        off = s * stripe + k * CHUNK
        pltpu.sync_copy(acc_sh.at[pl.ds(off, CHUNK)], b0)
        pltpu.sync_copy(b0, out_hbm.at[c, pl.ds(off, CHUNK)])


def _agg_sc(y, ei_pad, zeros2, p_rows):
    n_rows, width = y.shape
    t_chunks = ei_pad.shape[1] // NW
    assert t_chunks % 3 == 0
    body = functools.partial(_agg_body, t_chunks=t_chunks, p_rows=p_rows,
                             n_rows=n_rows, width=width)
    return pl.kernel(
        body,
        out_type=jax.ShapeDtypeStruct((NC, p_rows, width), jnp.float32),
        mesh=_MESH,
        compiler_params=pltpu.CompilerParams(use_tc_tiling_on_sc=False),
        scratch_types=[
            pltpu.VMEM_SHARED((p_rows, width), jnp.float32),
            pltpu.VMEM_SHARED((n_rows, width), jnp.float32),
            pltpu.VMEM((t_chunks, CHUNK), jnp.int32),
            pltpu.VMEM((t_chunks, CHUNK), jnp.int32),
        ] + [pltpu.VMEM((CHUNK, width), jnp.float32)] * 3
          + [pltpu.SemaphoreType.DMA] * 6,
    )(y, ei_pad, zeros2)


_BLK = 1000


def _mm_body(x_ref, w_ref, o_ref):
    o_ref[...] = jnp.dot(x_ref[...], w_ref[...],
                         preferred_element_type=jnp.float32)


def _matmul_tc(x, w):
    n, k = x.shape
    m = w.shape[1]
    grid = n // _BLK
    return pl.pallas_call(
        _mm_body,
        grid=(grid,),
        in_specs=[pl.BlockSpec((_BLK, k), lambda i: (i, 0)),
                  pl.BlockSpec((k, m), lambda i: (0, 0))],
        out_specs=pl.BlockSpec((_BLK, m), lambda i: (i, 0)),
        out_shape=jax.ShapeDtypeStruct((n, m), jnp.float32),
    )(x, w)


def _scale_body(xw_ref, degt_ref, y_ref, dinv_ref):
    deg = 1.0 + degt_ref[:, 0:1] + degt_ref[:, 1:2]
    dinv2 = 1.0 / jnp.sqrt(deg)
    y_ref[...] = dinv2 * xw_ref[...]
    dinv_ref[...] = dinv2


def _scale_tc(xw, degt):
    n, m = xw.shape
    grid = n // _BLK
    return pl.pallas_call(
        _scale_body,
        grid=(grid,),
        in_specs=[pl.BlockSpec((_BLK, m), lambda i: (i, 0)),
                  pl.BlockSpec((_BLK, NC), lambda i: (i, 0))],
        out_specs=[pl.BlockSpec((_BLK, m), lambda i: (i, 0)),
                   pl.BlockSpec((_BLK, 1), lambda i: (i, 0))],
        out_shape=[jax.ShapeDtypeStruct((n, m), jnp.float32),
                   jax.ShapeDtypeStruct((n, 1), jnp.float32)],
    )(xw, degt)


def _mid_body(y1_ref, p_ref, dinv_ref, b1_ref, wc_ref, y2_ref):
    agg = y1_ref[...] + p_ref[0] + p_ref[1]
    hidden = jnp.maximum(dinv_ref[...] * agg + b1_ref[...], 0.0)
    y2_ref[...] = dinv_ref[...] * jnp.dot(hidden, wc_ref[...],
                                          preferred_element_type=jnp.float32)


def _mid_tc(y1, parts, dinv, b1, wcat):
    n, m = y1.shape
    mo = wcat.shape[1]
    grid = n // _BLK
    return pl.pallas_call(
        _mid_body,
        grid=(grid,),
        in_specs=[pl.BlockSpec((_BLK, m), lambda i: (i, 0)),
                  pl.BlockSpec((NC, _BLK, m), lambda i: (0, i, 0)),
                  pl.BlockSpec((_BLK, 1), lambda i: (i, 0)),
                  pl.BlockSpec((1, m), lambda i: (0, 0)),
                  pl.BlockSpec((m, mo), lambda i: (0, 0))],
        out_specs=pl.BlockSpec((_BLK, mo), lambda i: (i, 0)),
        out_shape=jax.ShapeDtypeStruct((n, mo), jnp.float32),
    )(y1, parts, dinv, b1, wcat)


def _fin_body(y2_ref, p_ref, dinv_ref, bc_ref, eps_ref, z_ref, mu_ref,
              ls_ref, *, lat):
    agg = y2_ref[...] + p_ref[0] + p_ref[1]
    out = dinv_ref[...] * agg + bc_ref[...]
    mu = out[:, :lat]
    ls = out[:, lat:]
    mu_ref[...] = mu
    ls_ref[...] = ls
    z_ref[...] = mu + eps_ref[...] * jnp.exp(ls)


def _fin_tc(y2, parts, dinv, bcat, eps):
    n, m = y2.shape
    lat = m // 2
    grid = n // _BLK
    body = functools.partial(_fin_body, lat=lat)
    out_sd = jax.ShapeDtypeStruct((n, lat), jnp.float32)
    return pl.pallas_call(
        body,
        grid=(grid,),
        in_specs=[pl.BlockSpec((_BLK, m), lambda i: (i, 0)),
                  pl.BlockSpec((NC, _BLK, m), lambda i: (0, i, 0)),
                  pl.BlockSpec((_BLK, 1), lambda i: (i, 0)),
                  pl.BlockSpec((1, m), lambda i: (0, 0)),
                  pl.BlockSpec((_BLK, lat), lambda i: (i, 0))],
        out_specs=[pl.BlockSpec((_BLK, lat), lambda i: (i, 0)) for _ in range(3)],
        out_shape=[out_sd, out_sd, out_sd],
    )(y2, parts, dinv, bcat, eps)


@jax.jit
def kernel(x, edge_index, W1, b1, Wmu, bmu, Wls, bls, eps):
    n = x.shape[0]
    hid = W1.shape[1]
    p_rows = ((n + 1 + NS * CHUNK - 1) // (NS * CHUNK)) * (NS * CHUNK)

    ei_pad = _pad_edges(edge_index, n, p_rows)
    zeros1 = jnp.zeros((p_rows,), jnp.float32)
    zeros2 = jnp.zeros((p_rows, hid), jnp.float32)

    degp = _deg_sc(ei_pad, zeros1, p_rows).reshape(NC, p_rows)
    xw = _matmul_tc(x, W1)
    y1, dinv = _scale_tc(xw, degp[:, :n].T)

    p1 = _agg_sc(y1, ei_pad, zeros2, p_rows)
    wcat = jnp.concatenate([Wmu, Wls], axis=1)
    bcat = jnp.concatenate([bmu, bls])[None, :]
    y2 = _mid_tc(y1, p1, dinv, b1[None, :], wcat)

    p2 = _agg_sc(y2, ei_pad, zeros2, p_rows)
    z, mu, logstd = _fin_tc(y2, p2, dinv, bcat, eps)
    return (z, mu, logstd)


# speedup vs baseline: 1.3922x; 1.3922x over previous
import functools

import jax
import jax.numpy as jnp
from jax import lax
from jax.experimental import pallas as pl
from jax.experimental.pallas import tpu as pltpu
from jax.experimental.pallas import tpu_sc as plsc

NC = 2
NS = 16
NW = NC * NS
CHUNK = 128

_MESH = plsc.VectorSubcoreMesh(core_axis_name="c", subcore_axis_name="s")


def _pad_edges(edge_index, n_nodes, p_rows):
    e = edge_index.shape[1]
    per = NW * CHUNK * 3
    e_pad = ((e + per - 1) // per) * per
    extra = e_pad - e
    if extra:
        gar = n_nodes + jnp.arange(extra, dtype=jnp.int32) % (p_rows - n_nodes)
        pad = jnp.stack([jnp.zeros((extra,), jnp.int32), gar])
        edge_index = jnp.concatenate([edge_index, pad], axis=1)
    return edge_index.reshape(2, -1, CHUNK)


def _deg_body(ei_hbm, zeros_hbm, out_hbm, acc_sh, dst_sta, ones_v, sem, *,
              t_chunks, p_rows):
    c = lax.axis_index("c")
    s = lax.axis_index("s")
    stripe = p_rows // NS
    row0 = (c * NS + s) * t_chunks
    pltpu.sync_copy(ei_hbm.at[1, pl.ds(row0, t_chunks)], dst_sta)
    for k in range(stripe // CHUNK):
        off = s * stripe + k * CHUNK
        pltpu.sync_copy(zeros_hbm.at[pl.ds(off, CHUNK)], ones_v)
        pltpu.sync_copy(ones_v, acc_sh.at[pl.ds(off, CHUNK)])
    for i in range(CHUNK // 16):
        ones_v[pl.ds(i * 16, 16)] = jnp.ones((16,), jnp.float32)
    plsc.subcore_barrier()

    def body(j, carry):
        pltpu.async_copy(ones_v, acc_sh.at[dst_sta.at[j]], sem, add=True)
        return carry

    lax.fori_loop(0, t_chunks, body, 0)

    def drain(j, carry):
        pltpu.make_async_copy(ones_v, acc_sh.at[dst_sta.at[0]], sem).wait()
        return carry

    lax.fori_loop(0, t_chunks, drain, 0)
    plsc.subcore_barrier()
    zbuf = ones_v
    for k in range(stripe // CHUNK):
        off = s * stripe + k * CHUNK
        pltpu.sync_copy(acc_sh.at[pl.ds(off, CHUNK)], zbuf)
        pltpu.sync_copy(zbuf, out_hbm.at[pl.ds(c * p_rows + off, CHUNK)])


def _deg_sc(ei_pad, zeros1, p_rows):
    t_chunks = ei_pad.shape[1] // NW
    body = functools.partial(_deg_body, t_chunks=t_chunks, p_rows=p_rows)
    return pl.kernel(
        body,
        out_type=jax.ShapeDtypeStruct((NC * p_rows,), jnp.float32),
        mesh=_MESH,
        compiler_params=pltpu.CompilerParams(use_tc_tiling_on_sc=False),
        scratch_types=[
            pltpu.VMEM_SHARED((p_rows,), jnp.float32),
            pltpu.VMEM((t_chunks, CHUNK), jnp.int32),
            pltpu.VMEM((CHUNK,), jnp.float32),
            pltpu.SemaphoreType.DMA,
        ],
    )(ei_pad, zeros1)


def _agg_body(y_hbm, ei_hbm, zeros_hbm, out_hbm, acc_sh, y_sh, src_sta,
              dst_sta, b0, b1, b2, g0, g1, g2, s0, s1, s2, *,
              t_chunks, p_rows, n_rows, width):
    c = lax.axis_index("c")
    s = lax.axis_index("s")
    stripe = p_rows // NS
    row0 = s * t_chunks
    pltpu.sync_copy(ei_hbm.at[0, pl.ds(row0, t_chunks)], src_sta)
    pltpu.sync_copy(ei_hbm.at[1, pl.ds(row0, t_chunks)], dst_sta)
    for k in range(stripe // CHUNK):
        off = s * stripe + k * CHUNK
        pltpu.sync_copy(zeros_hbm.at[pl.ds(off, CHUNK)], b0)
        pltpu.sync_copy(b0, acc_sh.at[pl.ds(off, CHUNK)])
    y_stripe = n_rows // NS
    for k in range((y_stripe + CHUNK - 1) // CHUNK):
        off = s * y_stripe + k * CHUNK
        cnt = min(CHUNK, y_stripe - k * CHUNK)
        pltpu.sync_copy(y_hbm.at[c, pl.ds(off, cnt)], b1.at[pl.ds(0, cnt)])
        pltpu.sync_copy(b1.at[pl.ds(0, cnt)], y_sh.at[pl.ds(off, cnt)])
    plsc.subcore_barrier()

    bufs = (b0, b1, b2)
    gsem = (g0, g1, g2)
    ssem = (s0, s1, s2)

    def gather(q, b):
        pltpu.async_copy(y_sh.at[src_sta.at[q]], bufs[b], gsem[b])

    def gather_wait(b):
        pltpu.make_async_copy(y_sh.at[src_sta.at[0]], bufs[b],
                              gsem[b]).wait()

    def scatter(q, b):
        pltpu.async_copy(bufs[b], acc_sh.at[dst_sta.at[q]], ssem[b],
                         add=True)

    def scatter_wait(b):
        pltpu.make_async_copy(bufs[b], acc_sh.at[dst_sta.at[0]],
                              ssem[b]).wait()

    gather(0, 0)
    gather(1, 1)

    def body(i, carry):
        for b3 in range(3):
            q = i * 3 + b3
            nb = (b3 + 2) % 3

            @pl.when(q >= 1)
            def _():
                scatter_wait(nb)

            @pl.when(q + 2 < t_chunks)
            def _():
                gather(q + 2, nb)

            gather_wait(b3)
            scatter(q, b3)
        return carry

    lax.fori_loop(0, t_chunks // 3, body, 0)
    scatter_wait((t_chunks - 1) % 3)
    plsc.subcore_barrier()
    for k in range(stripe // CHUNK):
        off = s * stripe + k * CHUNK
        pltpu.sync_copy(acc_sh.at[pl.ds(off, CHUNK)], b0)
        pltpu.sync_copy(b0, out_hbm.at[c, pl.ds(off, CHUNK)])


def _agg_sc(y3, ei_pad, zeros2, p_rows):
    _, n_rows, width = y3.shape
    t_chunks = ei_pad.shape[1] // NS
    assert t_chunks % 3 == 0
    body = functools.partial(_agg_body, t_chunks=t_chunks, p_rows=p_rows,
                             n_rows=n_rows, width=width)
    return pl.kernel(
        body,
        out_type=jax.ShapeDtypeStruct((NC, p_rows, width), jnp.float32),
        mesh=_MESH,
        compiler_params=pltpu.CompilerParams(use_tc_tiling_on_sc=False),
        scratch_types=[
            pltpu.VMEM_SHARED((p_rows, width), jnp.float32),
            pltpu.VMEM_SHARED((n_rows, width), jnp.float32),
            pltpu.VMEM((t_chunks, CHUNK), jnp.int32),
            pltpu.VMEM((t_chunks, CHUNK), jnp.int32),
        ] + [pltpu.VMEM((CHUNK, width), jnp.float32)] * 3
          + [pltpu.SemaphoreType.DMA] * 6,
    )(y3, ei_pad, zeros2)


_BLK = 1000


def _mm_body(x_ref, w_ref, o_ref):
    o_ref[...] = jnp.dot(x_ref[...], w_ref[...],
                         preferred_element_type=jnp.float32)


def _matmul_tc(x, w):
    n, k = x.shape
    m = w.shape[1]
    grid = n // _BLK
    return pl.pallas_call(
        _mm_body,
        grid=(grid,),
        in_specs=[pl.BlockSpec((_BLK, k), lambda i: (i, 0)),
                  pl.BlockSpec((k, m), lambda i: (0, 0))],
        out_specs=pl.BlockSpec((_BLK, m), lambda i: (i, 0)),
        out_shape=jax.ShapeDtypeStruct((n, m), jnp.float32),
    )(x, w)


def _scale_body(xw_ref, degt_ref, y_ref, dinv_ref, *, half):
    deg = 1.0 + degt_ref[:, 0:1] + degt_ref[:, 1:2]
    dinv2 = 1.0 / jnp.sqrt(deg)
    y = dinv2 * xw_ref[...]
    y_ref[0] = y[:, :half]
    y_ref[1] = y[:, half:]
    dinv_ref[...] = dinv2


def _scale_tc(xw, degt):
    n, m = xw.shape
    half = m // 2
    grid = n // _BLK
    return pl.pallas_call(
        functools.partial(_scale_body, half=half),
        grid=(grid,),
        in_specs=[pl.BlockSpec((_BLK, m), lambda i: (i, 0)),
                  pl.BlockSpec((_BLK, NC), lambda i: (i, 0))],
        out_specs=[pl.BlockSpec((NC, _BLK, half), lambda i: (0, i, 0)),
                   pl.BlockSpec((_BLK, 1), lambda i: (i, 0))],
        out_shape=[jax.ShapeDtypeStruct((NC, n, half), jnp.float32),
                   jax.ShapeDtypeStruct((n, 1), jnp.float32)],
    )(xw, degt)


def _mid_body(y1_ref, p_ref, dinv_ref, b1_ref, wc_ref, y2_ref, *, half):
    agg = jnp.concatenate([y1_ref[0] + p_ref[0], y1_ref[1] + p_ref[1]],
                          axis=1)
    hidden = jnp.maximum(dinv_ref[...] * agg + b1_ref[...], 0.0)
    y2 = dinv_ref[...] * jnp.dot(hidden, wc_ref[...],
                                 preferred_element_type=jnp.float32)
    y2_ref[0] = y2[:, :half]
    y2_ref[1] = y2[:, half:]


def _mid_tc(y1s, parts, dinv, b1, wcat):
    _, n, half = y1s.shape
    m = wcat.shape[0]
    mo = wcat.shape[1]
    ho = mo // 2
    grid = n // _BLK
    return pl.pallas_call(
        functools.partial(_mid_body, half=ho),
        grid=(grid,),
        in_specs=[pl.BlockSpec((NC, _BLK, half), lambda i: (0, i, 0)),
                  pl.BlockSpec((NC, _BLK, half), lambda i: (0, i, 0)),
                  pl.BlockSpec((_BLK, 1), lambda i: (i, 0)),
                  pl.BlockSpec((1, m), lambda i: (0, 0)),
                  pl.BlockSpec((m, mo), lambda i: (0, 0))],
        out_specs=pl.BlockSpec((NC, _BLK, ho), lambda i: (0, i, 0)),
        out_shape=jax.ShapeDtypeStruct((NC, n, ho), jnp.float32),
    )(y1s, parts, dinv, b1, wcat)


def _fin_body(y2_ref, p_ref, dinv_ref, bc_ref, eps_ref, z_ref, mu_ref,
              ls_ref, *, lat):
    mu = dinv_ref[...] * (y2_ref[0] + p_ref[0]) + bc_ref[...][:, :lat]
    ls = dinv_ref[...] * (y2_ref[1] + p_ref[1]) + bc_ref[...][:, lat:]
    mu_ref[...] = mu
    ls_ref[...] = ls
    z_ref[...] = mu + eps_ref[...] * jnp.exp(ls)


def _fin_tc(y2s, parts, dinv, bcat, eps):
    _, n, lat = y2s.shape
    grid = n // _BLK
    body = functools.partial(_fin_body, lat=lat)
    out_sd = jax.ShapeDtypeStruct((n, lat), jnp.float32)
    return pl.pallas_call(
        body,
        grid=(grid,),
        in_specs=[pl.BlockSpec((NC, _BLK, lat), lambda i: (0, i, 0)),
                  pl.BlockSpec((NC, _BLK, lat), lambda i: (0, i, 0)),
                  pl.BlockSpec((_BLK, 1), lambda i: (i, 0)),
                  pl.BlockSpec((1, 2 * lat), lambda i: (0, 0)),
                  pl.BlockSpec((_BLK, lat), lambda i: (i, 0))],
        out_specs=[pl.BlockSpec((_BLK, lat), lambda i: (i, 0)) for _ in range(3)],
        out_shape=[out_sd, out_sd, out_sd],
    )(y2s, parts, dinv, bcat, eps)


@jax.jit
def kernel(x, edge_index, W1, b1, Wmu, bmu, Wls, bls, eps):
    n = x.shape[0]
    hid = W1.shape[1]
    p_rows = ((n + 1 + NS * CHUNK - 1) // (NS * CHUNK)) * (NS * CHUNK)

    ei_pad = _pad_edges(edge_index, n, p_rows)
    zeros1 = jnp.zeros((p_rows,), jnp.float32)
    zeros2 = jnp.zeros((p_rows, hid // 2), jnp.float32)

    degp = _deg_sc(ei_pad, zeros1, p_rows).reshape(NC, p_rows)
    xw = _matmul_tc(x, W1)
    y1s, dinv = _scale_tc(xw, degp[:, :n].T)

    p1 = _agg_sc(y1s, ei_pad, zeros2, p_rows)
    wcat = jnp.concatenate([Wmu, Wls], axis=1)
    bcat = jnp.concatenate([bmu, bls])[None, :]
    y2s = _mid_tc(y1s, p1, dinv, b1[None, :], wcat)

    p2 = _agg_sc(y2s, ei_pad, zeros2, p_rows)
    z, mu, logstd = _fin_tc(y2s, p2, dinv, bcat, eps)
    return (z, mu, logstd)


# speedup vs baseline: 1.4229x; 1.0220x over previous
import functools

import jax
import jax.numpy as jnp
from jax import lax
from jax.experimental import pallas as pl
from jax.experimental.pallas import tpu as pltpu
from jax.experimental.pallas import tpu_sc as plsc

NC = 2
NS = 16
NW = NC * NS
CHUNK = 128

_MESH = plsc.VectorSubcoreMesh(core_axis_name="c", subcore_axis_name="s")


def _pad_edges(edge_index, n_nodes, p_rows):
    e = edge_index.shape[1]
    per = NW * CHUNK * 3
    e_pad = ((e + per - 1) // per) * per
    extra = e_pad - e
    if extra:
        gar = n_nodes + jnp.arange(extra, dtype=jnp.int32) % (p_rows - n_nodes)
        pad = jnp.stack([jnp.zeros((extra,), jnp.int32), gar])
        edge_index = jnp.concatenate([edge_index, pad], axis=1)
    return edge_index.reshape(2, -1, CHUNK)


def _deg_body(ei_hbm, zeros_hbm, out_hbm, acc_sh, dst_sta, ones_v, sem, *,
              t_chunks, p_rows):
    c = lax.axis_index("c")
    s = lax.axis_index("s")
    stripe = p_rows // NS
    row0 = (c * NS + s) * t_chunks
    pltpu.sync_copy(ei_hbm.at[1, pl.ds(row0, t_chunks)], dst_sta)
    for k in range(stripe // CHUNK):
        off = s * stripe + k * CHUNK
        pltpu.sync_copy(zeros_hbm.at[pl.ds(off, CHUNK)], ones_v)
        pltpu.sync_copy(ones_v, acc_sh.at[pl.ds(off, CHUNK)])
    for i in range(CHUNK // 16):
        ones_v[pl.ds(i * 16, 16)] = jnp.ones((16,), jnp.float32)
    plsc.subcore_barrier()

    def body(j, carry):
        pltpu.async_copy(ones_v, acc_sh.at[dst_sta.at[j]], sem, add=True)
        return carry

    lax.fori_loop(0, t_chunks, body, 0)

    def drain(j, carry):
        pltpu.make_async_copy(ones_v, acc_sh.at[dst_sta.at[0]], sem).wait()
        return carry

    lax.fori_loop(0, t_chunks, drain, 0)
    plsc.subcore_barrier()
    zbuf = ones_v
    for k in range(stripe // CHUNK):
        off = s * stripe + k * CHUNK
        pltpu.sync_copy(acc_sh.at[pl.ds(off, CHUNK)], zbuf)
        pltpu.sync_copy(zbuf, out_hbm.at[pl.ds(c * p_rows + off, CHUNK)])


def _deg_sc(ei_pad, zeros1, p_rows):
    t_chunks = ei_pad.shape[1] // NW
    body = functools.partial(_deg_body, t_chunks=t_chunks, p_rows=p_rows)
    return pl.kernel(
        body,
        out_type=jax.ShapeDtypeStruct((NC * p_rows,), jnp.float32),
        mesh=_MESH,
        compiler_params=pltpu.CompilerParams(use_tc_tiling_on_sc=False),
        scratch_types=[
            pltpu.VMEM_SHARED((p_rows,), jnp.float32),
            pltpu.VMEM((t_chunks, CHUNK), jnp.int32),
            pltpu.VMEM((CHUNK,), jnp.float32),
            pltpu.SemaphoreType.DMA,
        ],
    )(ei_pad, zeros1)


def _agg_body(y_hbm, ei_hbm, zeros_hbm, out_hbm, acc_sh, y_sh, src_sta,
              dst_sta, b0, b1, b2, g0, g1, g2, s0, s1, s2, *,
              t_chunks, p_rows, n_rows, width):
    c = lax.axis_index("c")
    s = lax.axis_index("s")
    stripe = p_rows // NS
    row0 = s * t_chunks
    pltpu.sync_copy(ei_hbm.at[0, pl.ds(row0, t_chunks)], src_sta)
    pltpu.sync_copy(ei_hbm.at[1, pl.ds(row0, t_chunks)], dst_sta)
    for k in range(stripe // CHUNK):
        off = s * stripe + k * CHUNK
        pltpu.sync_copy(zeros_hbm.at[pl.ds(off, CHUNK)], b0)
        pltpu.sync_copy(b0, acc_sh.at[pl.ds(off, CHUNK)])
    y_stripe = n_rows // NS
    for k in range((y_stripe + CHUNK - 1) // CHUNK):
        off = s * y_stripe + k * CHUNK
        cnt = min(CHUNK, y_stripe - k * CHUNK)
        pltpu.sync_copy(y_hbm.at[c, pl.ds(off, cnt)], b1.at[pl.ds(0, cnt)])
        pltpu.sync_copy(b1.at[pl.ds(0, cnt)], y_sh.at[pl.ds(off, cnt)])
    plsc.subcore_barrier()

    bufs = (b0, b1, b2)
    gsem = (g0, g1, g2)
    ssem = (s0, s1, s2)

    def gather(q, b):
        pltpu.async_copy(y_sh.at[src_sta.at[q]], bufs[b], gsem[b])

    def gather_wait(b):
        pltpu.make_async_copy(y_sh.at[src_sta.at[0]], bufs[b],
                              gsem[b]).wait()

    def scatter(q, b):
        pltpu.async_copy(bufs[b], acc_sh.at[dst_sta.at[q]], ssem[b],
                         add=True)

    def scatter_wait(b):
        pltpu.make_async_copy(bufs[b], acc_sh.at[dst_sta.at[0]],
                              ssem[b]).wait()

    gather(0, 0)
    gather(1, 1)

    def body(i, carry):
        for b3 in range(3):
            q = i * 3 + b3
            nb = (b3 + 2) % 3

            @pl.when(q >= 1)
            def _():
                scatter_wait(nb)

            @pl.when(q + 2 < t_chunks)
            def _():
                gather(q + 2, nb)

            gather_wait(b3)
            scatter(q, b3)
        return carry

    lax.fori_loop(0, t_chunks // 3, body, 0)
    scatter_wait((t_chunks - 1) % 3)
    plsc.subcore_barrier()
    for k in range(stripe // CHUNK):
        off = s * stripe + k * CHUNK
        pltpu.sync_copy(acc_sh.at[pl.ds(off, CHUNK)], b0)
        pltpu.sync_copy(b0, out_hbm.at[c, pl.ds(off, CHUNK)])


def _agg_sc(y3, ei_pad, zeros2, p_rows):
    _, n_rows, width = y3.shape
    t_chunks = ei_pad.shape[1] // NS
    assert t_chunks % 3 == 0
    body = functools.partial(_agg_body, t_chunks=t_chunks, p_rows=p_rows,
                             n_rows=n_rows, width=width)
    return pl.kernel(
        body,
        out_type=jax.ShapeDtypeStruct((NC, p_rows, width), jnp.float32),
        mesh=_MESH,
        compiler_params=pltpu.CompilerParams(use_tc_tiling_on_sc=False),
        scratch_types=[
            pltpu.VMEM_SHARED((p_rows, width), jnp.float32),
            pltpu.VMEM_SHARED((n_rows, width), jnp.float32),
            pltpu.VMEM((t_chunks, CHUNK), jnp.int32),
            pltpu.VMEM((t_chunks, CHUNK), jnp.int32),
        ] + [pltpu.VMEM((CHUNK, width), jnp.float32)] * 3
          + [pltpu.SemaphoreType.DMA] * 6,
    )(y3, ei_pad, zeros2)


_BLK = 2000


def _mm_body(x_ref, w_ref, o_ref):
    o_ref[...] = jnp.dot(x_ref[...], w_ref[...],
                         preferred_element_type=jnp.float32)


def _matmul_tc(x, w):
    n, k = x.shape
    m = w.shape[1]
    grid = n // _BLK
    return pl.pallas_call(
        _mm_body,
        grid=(grid,),
        in_specs=[pl.BlockSpec((_BLK, k), lambda i: (i, 0)),
                  pl.BlockSpec((k, m), lambda i: (0, 0))],
        out_specs=pl.BlockSpec((_BLK, m), lambda i: (i, 0)),
        out_shape=jax.ShapeDtypeStruct((n, m), jnp.float32),
    )(x, w)


def _scale_body(xw_ref, degt_ref, y_ref, dinv_ref, *, half):
    deg = 1.0 + degt_ref[:, 0:1] + degt_ref[:, 1:2]
    dinv2 = 1.0 / jnp.sqrt(deg)
    y = dinv2 * xw_ref[...]
    y_ref[0] = y[:, :half]
    y_ref[1] = y[:, half:]
    dinv_ref[...] = dinv2


def _scale_tc(xw, degt):
    n, m = xw.shape
    half = m // 2
    grid = n // _BLK
    return pl.pallas_call(
        functools.partial(_scale_body, half=half),
        grid=(grid,),
        in_specs=[pl.BlockSpec((_BLK, m), lambda i: (i, 0)),
                  pl.BlockSpec((_BLK, NC), lambda i: (i, 0))],
        out_specs=[pl.BlockSpec((NC, _BLK, half), lambda i: (0, i, 0)),
                   pl.BlockSpec((_BLK, 1), lambda i: (i, 0))],
        out_shape=[jax.ShapeDtypeStruct((NC, n, half), jnp.float32),
                   jax.ShapeDtypeStruct((n, 1), jnp.float32)],
    )(xw, degt)


def _mid_body(y1_ref, p_ref, dinv_ref, b1_ref, wc_ref, y2_ref, *, half):
    agg = jnp.concatenate([y1_ref[0] + p_ref[0], y1_ref[1] + p_ref[1]],
                          axis=1)
    hidden = jnp.maximum(dinv_ref[...] * agg + b1_ref[...], 0.0)
    y2 = dinv_ref[...] * jnp.dot(hidden, wc_ref[...],
                                 preferred_element_type=jnp.float32)
    y2_ref[0] = y2[:, :half]
    y2_ref[1] = y2[:, half:]


def _mid_tc(y1s, parts, dinv, b1, wcat):
    _, n, half = y1s.shape
    m = wcat.shape[0]
    mo = wcat.shape[1]
    ho = mo // 2
    grid = n // _BLK
    return pl.pallas_call(
        functools.partial(_mid_body, half=ho),
        grid=(grid,),
        in_specs=[pl.BlockSpec((NC, _BLK, half), lambda i: (0, i, 0)),
                  pl.BlockSpec((NC, _BLK, half), lambda i: (0, i, 0)),
                  pl.BlockSpec((_BLK, 1), lambda i: (i, 0)),
                  pl.BlockSpec((1, m), lambda i: (0, 0)),
                  pl.BlockSpec((m, mo), lambda i: (0, 0))],
        out_specs=pl.BlockSpec((NC, _BLK, ho), lambda i: (0, i, 0)),
        out_shape=jax.ShapeDtypeStruct((NC, n, ho), jnp.float32),
    )(y1s, parts, dinv, b1, wcat)


def _fin_body(y2_ref, p_ref, dinv_ref, bc_ref, eps_ref, z_ref, mu_ref,
              ls_ref, *, lat):
    mu = dinv_ref[...] * (y2_ref[0] + p_ref[0]) + bc_ref[...][:, :lat]
    ls = dinv_ref[...] * (y2_ref[1] + p_ref[1]) + bc_ref[...][:, lat:]
    mu_ref[...] = mu
    ls_ref[...] = ls
    z_ref[...] = mu + eps_ref[...] * jnp.exp(ls)


def _fin_tc(y2s, parts, dinv, bcat, eps):
    _, n, lat = y2s.shape
    grid = n // _BLK
    body = functools.partial(_fin_body, lat=lat)
    out_sd = jax.ShapeDtypeStruct((n, lat), jnp.float32)
    return pl.pallas_call(
        body,
        grid=(grid,),
        in_specs=[pl.BlockSpec((NC, _BLK, lat), lambda i: (0, i, 0)),
                  pl.BlockSpec((NC, _BLK, lat), lambda i: (0, i, 0)),
                  pl.BlockSpec((_BLK, 1), lambda i: (i, 0)),
                  pl.BlockSpec((1, 2 * lat), lambda i: (0, 0)),
                  pl.BlockSpec((_BLK, lat), lambda i: (i, 0))],
        out_specs=[pl.BlockSpec((_BLK, lat), lambda i: (i, 0)) for _ in range(3)],
        out_shape=[out_sd, out_sd, out_sd],
    )(y2s, parts, dinv, bcat, eps)


@jax.jit
def kernel(x, edge_index, W1, b1, Wmu, bmu, Wls, bls, eps):
    n = x.shape[0]
    hid = W1.shape[1]
    p_rows = ((n + 1 + NS * CHUNK - 1) // (NS * CHUNK)) * (NS * CHUNK)

    ei_pad = _pad_edges(edge_index, n, p_rows)
    zeros1 = jnp.zeros((p_rows,), jnp.float32)
    zeros2 = jnp.zeros((p_rows, hid // 2), jnp.float32)

    degp = _deg_sc(ei_pad, zeros1, p_rows).reshape(NC, p_rows)
    xw = _matmul_tc(x, W1)
    y1s, dinv = _scale_tc(xw, degp[:, :n].T)

    p1 = _agg_sc(y1s, ei_pad, zeros2, p_rows)
    wcat = jnp.concatenate([Wmu, Wls], axis=1)
    bcat = jnp.concatenate([bmu, bls])[None, :]
    y2s = _mid_tc(y1s, p1, dinv, b1[None, :], wcat)

    p2 = _agg_sc(y2s, ei_pad, zeros2, p_rows)
    z, mu, logstd = _fin_tc(y2s, p2, dinv, bcat, eps)
    return (z, mu, logstd)


# speedup vs baseline: 1.4298x; 1.0049x over previous
import functools

import jax
import jax.numpy as jnp
from jax import lax
from jax.experimental import pallas as pl
from jax.experimental.pallas import tpu as pltpu
from jax.experimental.pallas import tpu_sc as plsc

NC = 2
NS = 16
NW = NC * NS
CHUNK = 128

_MESH = plsc.VectorSubcoreMesh(core_axis_name="c", subcore_axis_name="s")


def _pad_edges(edge_index, n_nodes, p_rows):
    e = edge_index.shape[1]
    per = NW * CHUNK * 3
    e_pad = ((e + per - 1) // per) * per
    extra = e_pad - e
    if extra:
        gar = n_nodes + jnp.arange(extra, dtype=jnp.int32) % (p_rows - n_nodes)
        pad = jnp.stack([jnp.zeros((extra,), jnp.int32), gar])
        edge_index = jnp.concatenate([edge_index, pad], axis=1)
    return edge_index.reshape(2, -1, CHUNK)


def _deg_body(ei_hbm, zeros_hbm, out_hbm, acc_sh, dst_sta, ones_v, sem, *,
              t_chunks, p_rows):
    c = lax.axis_index("c")
    s = lax.axis_index("s")
    stripe = p_rows // NS
    row0 = (c * NS + s) * t_chunks
    pltpu.sync_copy(ei_hbm.at[1, pl.ds(row0, t_chunks)], dst_sta)
    for k in range(stripe // CHUNK):
        off = s * stripe + k * CHUNK
        pltpu.sync_copy(zeros_hbm.at[pl.ds(off, CHUNK)], ones_v)
        pltpu.sync_copy(ones_v, acc_sh.at[pl.ds(off, CHUNK)])
    for i in range(CHUNK // 16):
        ones_v[pl.ds(i * 16, 16)] = jnp.ones((16,), jnp.float32)
    plsc.subcore_barrier()

    def body(j, carry):
        pltpu.async_copy(ones_v, acc_sh.at[dst_sta.at[j]], sem, add=True)
        return carry

    lax.fori_loop(0, t_chunks, body, 0)

    def drain(j, carry):
        pltpu.make_async_copy(ones_v, acc_sh.at[dst_sta.at[0]], sem).wait()
        return carry

    lax.fori_loop(0, t_chunks, drain, 0)
    plsc.subcore_barrier()
    zbuf = ones_v
    for k in range(stripe // CHUNK):
        off = s * stripe + k * CHUNK
        pltpu.sync_copy(acc_sh.at[pl.ds(off, CHUNK)], zbuf)
        pltpu.sync_copy(zbuf, out_hbm.at[pl.ds(c * p_rows + off, CHUNK)])


def _deg_sc(ei_pad, zeros1, p_rows):
    t_chunks = ei_pad.shape[1] // NW
    body = functools.partial(_deg_body, t_chunks=t_chunks, p_rows=p_rows)
    return pl.kernel(
        body,
        out_type=jax.ShapeDtypeStruct((NC * p_rows,), jnp.float32),
        mesh=_MESH,
        compiler_params=pltpu.CompilerParams(use_tc_tiling_on_sc=False),
        scratch_types=[
            pltpu.VMEM_SHARED((p_rows,), jnp.float32),
            pltpu.VMEM((t_chunks, CHUNK), jnp.int32),
            pltpu.VMEM((CHUNK,), jnp.float32),
            pltpu.SemaphoreType.DMA,
        ],
    )(ei_pad, zeros1)


def _agg_body(y_hbm, ei_hbm, zeros_hbm, out_hbm, acc_sh, y_sh, src_sta,
              dst_sta, b0, b1, b2, g0, g1, g2, s0, s1, s2, *,
              t_chunks, p_rows, n_rows, width):
    c = lax.axis_index("c")
    s = lax.axis_index("s")
    stripe = p_rows // NS
    row0 = s * t_chunks
    pltpu.sync_copy(ei_hbm.at[0, pl.ds(row0, t_chunks)], src_sta)
    pltpu.sync_copy(ei_hbm.at[1, pl.ds(row0, t_chunks)], dst_sta)
    for k in range(stripe // CHUNK):
        off = s * stripe + k * CHUNK
        pltpu.sync_copy(zeros_hbm.at[pl.ds(off, CHUNK)], b0)
        pltpu.sync_copy(b0, acc_sh.at[pl.ds(off, CHUNK)])
    y_stripe = n_rows // NS
    for k in range((y_stripe + CHUNK - 1) // CHUNK):
        off = s * y_stripe + k * CHUNK
        cnt = min(CHUNK, y_stripe - k * CHUNK)
        pltpu.sync_copy(y_hbm.at[c, pl.ds(off, cnt)], b1.at[pl.ds(0, cnt)])
        pltpu.sync_copy(b1.at[pl.ds(0, cnt)], y_sh.at[pl.ds(off, cnt)])
    plsc.subcore_barrier()

    bufs = (b0, b1, b2)
    gsem = (g0, g1, g2)
    ssem = (s0, s1, s2)

    def gather(q, b):
        pltpu.async_copy(y_sh.at[src_sta.at[q]], bufs[b], gsem[b])

    def gather_wait(b):
        pltpu.make_async_copy(y_sh.at[src_sta.at[0]], bufs[b],
                              gsem[b]).wait()

    def scatter(q, b):
        pltpu.async_copy(bufs[b], acc_sh.at[dst_sta.at[q]], ssem[b],
                         add=True)

    def scatter_wait(b):
        pltpu.make_async_copy(bufs[b], acc_sh.at[dst_sta.at[0]],
                              ssem[b]).wait()

    gather(0, 0)
    gather(1, 1)

    def body(i, carry):
        for b3 in range(3):
            q = i * 3 + b3
            nb = (b3 + 2) % 3

            @pl.when(q >= 1)
            def _():
                scatter_wait(nb)

            @pl.when(q + 2 < t_chunks)
            def _():
                gather(q + 2, nb)

            gather_wait(b3)
            scatter(q, b3)
        return carry

    lax.fori_loop(0, t_chunks // 3, body, 0)
    scatter_wait((t_chunks - 1) % 3)
    plsc.subcore_barrier()
    for k in range(stripe // CHUNK):
        off = s * stripe + k * CHUNK
        pltpu.sync_copy(acc_sh.at[pl.ds(off, CHUNK)], b0)
        pltpu.sync_copy(b0, out_hbm.at[c, pl.ds(off, CHUNK)])


def _agg_sc(y3, ei_pad, zeros2, p_rows):
    _, n_rows, width = y3.shape
    t_chunks = ei_pad.shape[1] // NS
    assert t_chunks % 3 == 0
    body = functools.partial(_agg_body, t_chunks=t_chunks, p_rows=p_rows,
                             n_rows=n_rows, width=width)
    return pl.kernel(
        body,
        out_type=jax.ShapeDtypeStruct((NC, p_rows, width), jnp.float32),
        mesh=_MESH,
        compiler_params=pltpu.CompilerParams(use_tc_tiling_on_sc=False),
        scratch_types=[
            pltpu.VMEM_SHARED((p_rows, width), jnp.float32),
            pltpu.VMEM_SHARED((n_rows, width), jnp.float32),
            pltpu.VMEM((t_chunks, CHUNK), jnp.int32),
            pltpu.VMEM((t_chunks, CHUNK), jnp.int32),
        ] + [pltpu.VMEM((CHUNK, width), jnp.float32)] * 3
          + [pltpu.SemaphoreType.DMA] * 6,
    )(y3, ei_pad, zeros2)


_BLK = 5000


def _mm_body(x_ref, w_ref, o_ref):
    o_ref[...] = jnp.dot(x_ref[...], w_ref[...],
                         preferred_element_type=jnp.float32)


def _matmul_tc(x, w):
    n, k = x.shape
    m = w.shape[1]
    grid = n // _BLK
    return pl.pallas_call(
        _mm_body,
        grid=(grid,),
        in_specs=[pl.BlockSpec((_BLK, k), lambda i: (i, 0)),
                  pl.BlockSpec((k, m), lambda i: (0, 0))],
        out_specs=pl.BlockSpec((_BLK, m), lambda i: (i, 0)),
        out_shape=jax.ShapeDtypeStruct((n, m), jnp.float32),
    )(x, w)


def _scale_body(xw_ref, degt_ref, y_ref, dinv_ref, *, half):
    deg = 1.0 + degt_ref[:, 0:1] + degt_ref[:, 1:2]
    dinv2 = 1.0 / jnp.sqrt(deg)
    y = dinv2 * xw_ref[...]
    y_ref[0] = y[:, :half]
    y_ref[1] = y[:, half:]
    dinv_ref[...] = dinv2


def _scale_tc(xw, degt):
    n, m = xw.shape
    half = m // 2
    grid = n // _BLK
    return pl.pallas_call(
        functools.partial(_scale_body, half=half),
        grid=(grid,),
        in_specs=[pl.BlockSpec((_BLK, m), lambda i: (i, 0)),
                  pl.BlockSpec((_BLK, NC), lambda i: (i, 0))],
        out_specs=[pl.BlockSpec((NC, _BLK, half), lambda i: (0, i, 0)),
                   pl.BlockSpec((_BLK, 1), lambda i: (i, 0))],
        out_shape=[jax.ShapeDtypeStruct((NC, n, half), jnp.float32),
                   jax.ShapeDtypeStruct((n, 1), jnp.float32)],
    )(xw, degt)


def _mid_body(y1_ref, p_ref, dinv_ref, b1_ref, wc_ref, y2_ref, *, half):
    agg = jnp.concatenate([y1_ref[0] + p_ref[0], y1_ref[1] + p_ref[1]],
                          axis=1)
    hidden = jnp.maximum(dinv_ref[...] * agg + b1_ref[...], 0.0)
    y2 = dinv_ref[...] * jnp.dot(hidden, wc_ref[...],
                                 preferred_element_type=jnp.float32)
    y2_ref[0] = y2[:, :half]
    y2_ref[1] = y2[:, half:]


def _mid_tc(y1s, parts, dinv, b1, wcat):
    _, n, half = y1s.shape
    m = wcat.shape[0]
    mo = wcat.shape[1]
    ho = mo // 2
    grid = n // _BLK
    return pl.pallas_call(
        functools.partial(_mid_body, half=ho),
        grid=(grid,),
        in_specs=[pl.BlockSpec((NC, _BLK, half), lambda i: (0, i, 0)),
                  pl.BlockSpec((NC, _BLK, half), lambda i: (0, i, 0)),
                  pl.BlockSpec((_BLK, 1), lambda i: (i, 0)),
                  pl.BlockSpec((1, m), lambda i: (0, 0)),
                  pl.BlockSpec((m, mo), lambda i: (0, 0))],
        out_specs=pl.BlockSpec((NC, _BLK, ho), lambda i: (0, i, 0)),
        out_shape=jax.ShapeDtypeStruct((NC, n, ho), jnp.float32),
    )(y1s, parts, dinv, b1, wcat)


def _fin_body(y2_ref, p_ref, dinv_ref, bc_ref, eps_ref, z_ref, mu_ref,
              ls_ref, *, lat):
    mu = dinv_ref[...] * (y2_ref[0] + p_ref[0]) + bc_ref[...][:, :lat]
    ls = dinv_ref[...] * (y2_ref[1] + p_ref[1]) + bc_ref[...][:, lat:]
    mu_ref[...] = mu
    ls_ref[...] = ls
    z_ref[...] = mu + eps_ref[...] * jnp.exp(ls)


def _fin_tc(y2s, parts, dinv, bcat, eps):
    _, n, lat = y2s.shape
    grid = n // _BLK
    body = functools.partial(_fin_body, lat=lat)
    out_sd = jax.ShapeDtypeStruct((n, lat), jnp.float32)
    return pl.pallas_call(
        body,
        grid=(grid,),
        in_specs=[pl.BlockSpec((NC, _BLK, lat), lambda i: (0, i, 0)),
                  pl.BlockSpec((NC, _BLK, lat), lambda i: (0, i, 0)),
                  pl.BlockSpec((_BLK, 1), lambda i: (i, 0)),
                  pl.BlockSpec((1, 2 * lat), lambda i: (0, 0)),
                  pl.BlockSpec((_BLK, lat), lambda i: (i, 0))],
        out_specs=[pl.BlockSpec((_BLK, lat), lambda i: (i, 0)) for _ in range(3)],
        out_shape=[out_sd, out_sd, out_sd],
    )(y2s, parts, dinv, bcat, eps)


@jax.jit
def kernel(x, edge_index, W1, b1, Wmu, bmu, Wls, bls, eps):
    n = x.shape[0]
    hid = W1.shape[1]
    p_rows = ((n + 1 + NS * CHUNK - 1) // (NS * CHUNK)) * (NS * CHUNK)

    ei_pad = _pad_edges(edge_index, n, p_rows)
    zeros1 = jnp.zeros((p_rows,), jnp.float32)
    zeros2 = jnp.zeros((p_rows, hid // 2), jnp.float32)

    degp = _deg_sc(ei_pad, zeros1, p_rows).reshape(NC, p_rows)
    xw = _matmul_tc(x, W1)
    y1s, dinv = _scale_tc(xw, degp[:, :n].T)

    p1 = _agg_sc(y1s, ei_pad, zeros2, p_rows)
    wcat = jnp.concatenate([Wmu, Wls], axis=1)
    bcat = jnp.concatenate([bmu, bls])[None, :]
    y2s = _mid_tc(y1s, p1, dinv, b1[None, :], wcat)

    p2 = _agg_sc(y2s, ei_pad, zeros2, p_rows)
    z, mu, logstd = _fin_tc(y2s, p2, dinv, bcat, eps)
    return (z, mu, logstd)
